# baseline (speedup 1.0000x reference)
import functools

import jax
import jax.numpy as jnp
from jax.experimental import pallas as pl
from jax.experimental.pallas import tpu as pltpu
from jax.experimental.pallas import tpu_sc as plsc

_EP = 2048
_C = 97
_CP = 128
_BIG = 1e30
_NC = 2
_NS = 16
_PPW = _EP // (_NC * _NS)
_L = 16


def _sc_gather_body(lp_hbm, starts_hbm, out_hbm, idxv, rowsv, sem):
    wid = jax.lax.axis_index("s") * _NC + jax.lax.axis_index("c")
    base = wid * _PPW
    pltpu.sync_copy(starts_hbm.at[pl.ds(base, _PPW)], idxv)
    pltpu.async_copy(lp_hbm.at[idxv], rowsv, sem).wait()
    pltpu.sync_copy(rowsv, out_hbm.at[pl.ds(base, _PPW)])


_sc_gather = functools.partial(
    pl.kernel,
    out_type=jax.ShapeDtypeStruct((_EP, _CP), jnp.float32),
    mesh=plsc.VectorSubcoreMesh(core_axis_name="c", subcore_axis_name="s"),
    scratch_types=[
        pltpu.VMEM((_PPW,), jnp.int32),
        pltpu.VMEM((_PPW, _CP), jnp.float32),
        pltpu.SemaphoreType.DMA,
    ],
)(_sc_gather_body)


def _loss_body(ep_ref, labels_ref, col0_ref, out_ref):
    e = ep_ref[...][:, :_C]
    lab = labels_ref[...]
    col = jax.lax.broadcasted_iota(jnp.int32, (_EP, _C), 1)
    isc0 = col == 0
    e = jnp.where(isc0, col0_ref[...], e)
    lab = jnp.where(isc0, 0.0, lab)
    th = isc0.astype(jnp.float32)

    p_mask = lab + th
    e1 = e - (1.0 - p_mask) * _BIG
    m1 = jnp.max(e1, axis=1, keepdims=True)
    lse1 = m1 + jnp.log(jnp.sum(jnp.exp(e1 - m1), axis=1, keepdims=True))
    loss1 = jnp.sum(lab * (lse1 - e1))

    e2 = e - lab * _BIG
    m2 = jnp.max(e2, axis=1, keepdims=True)
    lse2 = m2 + jnp.log(jnp.sum(jnp.exp(e2 - m2), axis=1, keepdims=True))
    loss2 = jnp.sum(lse2[:, 0] - e[:, 0])

    out_ref[...] = jnp.reshape((loss1 + loss2) * (1.0 / _EP), (1, 1))


def kernel(logits, labels, pos):
    starts = pos.astype(jnp.int32)[:, 0]
    lp = jnp.pad(logits, ((0, 0), (0, _CP - _C)))
    e_pad = _sc_gather(lp, starts)
    col0 = jax.lax.slice(logits, (0, 0), (_EP, 1))
    out = pl.pallas_call(
        _loss_body,
        out_shape=jax.ShapeDtypeStruct((1, 1), jnp.float32),
    )(e_pad, labels, col0)
    return out[0, 0]
